# baseline (speedup 1.0000x reference)
import jax
import jax.numpy as jnp
from jax import lax
from jax.experimental import pallas as pl
from jax.experimental.pallas import tpu as pltpu


def _topk_body(lp_ref, bias_ref, seq_ref, seq_out_ref, sum_out_ref, src_out_ref):
    nb = lp_ref.shape[1]
    v = lp_ref.shape[2]
    t = seq_ref.shape[2]
    b = pl.program_id(0)
    x = lp_ref[0] + bias_ref[0][:, 0:1]
    rowi = lax.broadcasted_iota(jnp.int32, (nb, v), 0)
    coli = lax.broadcasted_iota(jnp.int32, (nb, v), 1)
    flat = rowi * v + coli
    vals = jnp.zeros((1, nb), jnp.float32)
    srcs = jnp.zeros((1, nb), jnp.int32)
    selc = jnp.zeros((nb, 1), jnp.int32)
    prefix = jnp.zeros((nb, t), jnp.int32)
    li = lax.broadcasted_iota(jnp.int32, (1, nb), 1)
    ri = lax.broadcasted_iota(jnp.int32, (nb, 1), 0)
    rt = lax.broadcasted_iota(jnp.int32, (nb, t), 0)
    for k in range(nb):
        m = jnp.max(x)
        idx = jnp.min(jnp.where(x >= m, flat, jnp.int32(nb * v)))
        bix = idx // v
        sel = idx - bix * v
        seq_row = seq_ref[0, pl.ds(bix, 1), :]
        vals = jnp.where(li == k, m, vals)
        srcs = jnp.where(li == k, bix, srcs)
        selc = jnp.where(ri == k, sel, selc)
        prefix = jnp.where(rt == k, seq_row, prefix)
        x = jnp.where(flat == idx, -jnp.inf, x)
    sum_out_ref[0] = vals
    src_out_ref[0] = srcs + b * nb
    seq_out_ref[0] = jnp.concatenate([prefix, selc], axis=1).astype(seq_out_ref.dtype)


def _gather_body(src_ref, slp_ref, lp_ref, st_ref, oslp_ref, ost_ref):
    t = slp_ref.shape[1]
    nb = oslp_ref.shape[2]
    jj = pl.program_id(0) % nb
    oslp_ref[0, 0:t, pl.ds(jj, 1), :] = slp_ref[0]
    oslp_ref[0, t:t + 1, pl.ds(jj, 1), :] = lp_ref[...]
    ost_ref[:, 0, pl.ds(jj, 1), :] = st_ref[:, 0, :, :]


def kernel(logprobs, beam_logprobs_sum, beam_seq, beam_seq_logprobs, state):
    B, BD = beam_logprobs_sum.shape
    V = logprobs.shape[-1]
    T = beam_seq.shape[-1]
    L, R, D = state.shape

    lp3 = logprobs.reshape(B, BD, V)
    bias = jnp.broadcast_to(beam_logprobs_sum[:, :, None], (B, BD, 128))

    seq_out, sums, srcs = pl.pallas_call(
        _topk_body,
        grid=(B,),
        in_specs=[
            pl.BlockSpec((1, BD, V), lambda b: (b, 0, 0)),
            pl.BlockSpec((1, BD, 128), lambda b: (b, 0, 0)),
            pl.BlockSpec((1, BD, T), lambda b: (b, 0, 0)),
        ],
        out_specs=[
            pl.BlockSpec((1, BD, T + 1), lambda b: (b, 0, 0)),
            pl.BlockSpec((1, 1, BD), lambda b: (b, 0, 0)),
            pl.BlockSpec((1, 1, BD), lambda b: (b, 0, 0)),
        ],
        out_shape=[
            jax.ShapeDtypeStruct((B, BD, T + 1), beam_seq.dtype),
            jax.ShapeDtypeStruct((B, 1, BD), jnp.float32),
            jax.ShapeDtypeStruct((B, 1, BD), jnp.int32),
        ],
    )(lp3, bias, beam_seq)

    srcflat = srcs.reshape(-1)
    slp4 = beam_seq_logprobs.reshape(B * BD, T, 1, V)
    lp3r = logprobs.reshape(B * BD, 1, V)
    st4 = state.reshape(L, R, 1, D)

    grid_spec = pltpu.PrefetchScalarGridSpec(
        num_scalar_prefetch=1,
        grid=(B * BD,),
        in_specs=[
            pl.BlockSpec((1, T, 1, V), lambda i, s: (s[i], 0, 0, 0)),
            pl.BlockSpec((1, 1, V), lambda i, s: (s[i], 0, 0)),
            pl.BlockSpec((L, 1, 1, D), lambda i, s: (0, s[i], 0, 0)),
        ],
        out_specs=[
            pl.BlockSpec((1, T + 1, BD, V), lambda i, s: (i // BD, 0, 0, 0)),
            pl.BlockSpec((L, 1, BD, D), lambda i, s: (0, i // BD, 0, 0)),
        ],
    )
    oslp, ost = pl.pallas_call(
        _gather_body,
        grid_spec=grid_spec,
        out_shape=[
            jax.ShapeDtypeStruct((B, T + 1, BD, V), jnp.float32),
            jax.ShapeDtypeStruct((L, B, BD, D), jnp.float32),
        ],
    )(srcflat, slp4, lp3r, st4)

    return (seq_out,
            oslp.transpose(0, 2, 1, 3),
            sums.reshape(B, BD),
            ost.reshape(L, R, D))


# speedup vs baseline: 1.7691x; 1.7691x over previous
import jax
import jax.numpy as jnp
from jax import lax
from jax.experimental import pallas as pl
from jax.experimental.pallas import tpu as pltpu


def _topk_body(lp_ref, bias_ref, seq_ref, seq_out_ref, sum_out_ref, src_out_ref):
    nb = lp_ref.shape[1]
    v = lp_ref.shape[2]
    t = seq_ref.shape[2]
    b = pl.program_id(0)
    x = lp_ref[0] + bias_ref[0][:, 0:1]
    rowi = lax.broadcasted_iota(jnp.int32, (nb, v), 0)
    coli = lax.broadcasted_iota(jnp.int32, (nb, v), 1)
    flat = rowi * v + coli
    vals = jnp.zeros((1, nb), jnp.float32)
    srcs = jnp.zeros((1, nb), jnp.int32)
    selc = jnp.zeros((nb, 1), jnp.int32)
    prefix = jnp.zeros((nb, t), jnp.int32)
    li = lax.broadcasted_iota(jnp.int32, (1, nb), 1)
    ri = lax.broadcasted_iota(jnp.int32, (nb, 1), 0)
    rt = lax.broadcasted_iota(jnp.int32, (nb, t), 0)
    for k in range(nb):
        m = jnp.max(x)
        idx = jnp.min(jnp.where(x >= m, flat, jnp.int32(nb * v)))
        bix = idx // v
        sel = idx - bix * v
        seq_row = seq_ref[0, pl.ds(bix, 1), :]
        vals = jnp.where(li == k, m, vals)
        srcs = jnp.where(li == k, bix, srcs)
        selc = jnp.where(ri == k, sel, selc)
        prefix = jnp.where(rt == k, seq_row, prefix)
        x = jnp.where(flat == idx, -jnp.inf, x)
    sum_out_ref[0] = vals
    src_out_ref[0] = srcs + b * nb
    seq_out_ref[0] = jnp.concatenate([prefix, selc], axis=1).astype(seq_out_ref.dtype)


def _gather_body(src_ref, slp_ref, lp_ref, st_ref, oslp_ref, ost_ref):
    t = slp_ref.shape[1]
    nb = oslp_ref.shape[2]
    i = pl.program_id(0)
    jj = i % nb
    bix = src_ref[i] - (i // nb) * nb
    for tt in range(t):
        oslp_ref[0, tt, pl.ds(jj, 1), :] = slp_ref[0, pl.ds(tt, 1), :]
    oslp_ref[0, t, pl.ds(jj, 1), :] = lp_ref[pl.ds(bix, 1), :]
    ost_ref[:, 0, pl.ds(jj, 1), :] = st_ref[:, 0, pl.ds(bix, 1), :]


def kernel(logprobs, beam_logprobs_sum, beam_seq, beam_seq_logprobs, state):
    B, BD = beam_logprobs_sum.shape
    V = logprobs.shape[-1]
    T = beam_seq.shape[-1]
    L, R, D = state.shape

    lp3 = logprobs.reshape(B, BD, V)
    bias = jnp.broadcast_to(beam_logprobs_sum[:, :, None], (B, BD, 128))

    seq_out, sums, srcs = pl.pallas_call(
        _topk_body,
        grid=(B,),
        in_specs=[
            pl.BlockSpec((1, BD, V), lambda b: (b, 0, 0)),
            pl.BlockSpec((1, BD, 128), lambda b: (b, 0, 0)),
            pl.BlockSpec((1, BD, T), lambda b: (b, 0, 0)),
        ],
        out_specs=[
            pl.BlockSpec((1, BD, T + 1), lambda b: (b, 0, 0)),
            pl.BlockSpec((1, 1, BD), lambda b: (b, 0, 0)),
            pl.BlockSpec((1, 1, BD), lambda b: (b, 0, 0)),
        ],
        out_shape=[
            jax.ShapeDtypeStruct((B, BD, T + 1), beam_seq.dtype),
            jax.ShapeDtypeStruct((B, 1, BD), jnp.float32),
            jax.ShapeDtypeStruct((B, 1, BD), jnp.int32),
        ],
    )(lp3, bias, beam_seq)

    srcflat = srcs.reshape(-1)
    slp4 = beam_seq_logprobs.reshape(B * BD, T, V)
    st8 = state.reshape(L, B, BD, D)

    grid_spec = pltpu.PrefetchScalarGridSpec(
        num_scalar_prefetch=1,
        grid=(B * BD,),
        in_specs=[
            pl.BlockSpec((1, T, V), lambda i, s: (s[i], 0, 0)),
            pl.BlockSpec((BD, V), lambda i, s: (i // BD, 0)),
            pl.BlockSpec((L, 1, BD, D), lambda i, s: (0, i // BD, 0, 0)),
        ],
        out_specs=[
            pl.BlockSpec((1, T + 1, BD, V), lambda i, s: (i // BD, 0, 0, 0)),
            pl.BlockSpec((L, 1, BD, D), lambda i, s: (0, i // BD, 0, 0)),
        ],
    )
    oslp, ost = pl.pallas_call(
        _gather_body,
        grid_spec=grid_spec,
        out_shape=[
            jax.ShapeDtypeStruct((B, T + 1, BD, V), jnp.float32),
            jax.ShapeDtypeStruct((L, B, BD, D), jnp.float32),
        ],
    )(srcflat, slp4, logprobs, st8)

    return (seq_out,
            oslp.transpose(0, 2, 1, 3),
            sums.reshape(B, BD),
            ost.reshape(L, R, D))


# speedup vs baseline: 2.3636x; 1.3360x over previous
import functools

import jax
import jax.numpy as jnp
from jax import lax
from jax.experimental import pallas as pl
from jax.experimental.pallas import tpu as pltpu
from jax.experimental.pallas import tpu_sc as plsc

_NB = 8
_V = 32768
_CH = 8192
_LN = 16
_NCH = 32

_DNUMS = lax.GatherDimensionNumbers(
    offset_dims=(), collapsed_slice_dims=(0,), start_index_map=(0,))


def _sc_topk_body(lp_ref, bias_ref, osum_ref, osrc_ref, osel_ref,
                  buf0, buf1, biasv, tvv, tiv, thv,
                  stg_v, stg_src, stg_sel, sem0, sem1):
    w = lax.axis_index("s") * 2 + lax.axis_index("c")
    lane = lax.iota(jnp.int32, _LN)
    neg = jnp.float32(-jnp.inf)
    eight = jnp.full((_LN,), 8, jnp.int32)

    def lane_splat(vec, idx):
        return lax.gather(vec, idx[:, None], _DNUMS, (1,),
                          mode=lax.GatherScatterMode.PROMISE_IN_BOUNDS)

    tvv[...] = jnp.full((_LN,), neg, jnp.float32)
    tiv[...] = jnp.full((_LN,), 0, jnp.int32)
    thv[...] = jnp.full((_LN,), neg, jnp.float32)

    pltpu.sync_copy(bias_ref.at[pl.ds(w * (_NB * _LN), _NB * _LN)], biasv)

    def copy(t, buf, sem):
        row = w * _NB + t // 4
        off = (t % 4) * _CH
        return pltpu.make_async_copy(lp_ref.at[row, pl.ds(off, _CH)], buf, sem)

    def bias_vec(s):
        return biasv[pl.ds(s * _LN, _LN)]

    def merge(vb, fb):
        tv = tvv[...]
        ti = tiv[...]
        idx = jnp.full((_LN,), fb, jnp.int32) + lane
        cav, cai = plsc.sort_key_val(vb, idx, descending=False)
        cdv = lax.rev(cav, (0,))
        cdi = lax.rev(cai, (0,))
        lo = lane < 8
        combv = jnp.where(lo, cdv, tv)
        combi = jnp.where(lo, cdi, ti)
        tv2, ti2 = plsc.sort_key_val(combv, combi, descending=False)
        tvv[...] = tv2
        tiv[...] = ti2
        thv[...] = lane_splat(tv2, eight)

    def scan_chunk(t, buf):
        jt = t // 4
        voff = (t % 4) * _CH
        bv = bias_vec(jt)

        def sub(g, carry):
            base = g * 1024
            th0 = thv[...][0]
            rm = jnp.full((_LN,), neg, jnp.float32)
            for k in range(64):
                v = buf[pl.ds(base + k * _LN, _LN)]
                rm = jnp.maximum(rm, v)
            srt, _ = plsc.sort_key_val(rm + bv, lane, descending=False)

            @pl.when(srt[_LN - 1] > th0)
            def _rescan():
                def grp(g2, c2):
                    gb = base + g2 * 128
                    gm = jnp.full((_LN,), neg, jnp.float32)
                    for k in range(8):
                        gm = jnp.maximum(gm, buf[pl.ds(gb + k * _LN, _LN)])
                    gs, _ = plsc.sort_key_val(gm + bv, lane,
                                              descending=False)

                    @pl.when(gs[_LN - 1] > thv[...][0])
                    def _grp_scan():
                        def vec_one(k2, c3):
                            vb = buf[pl.ds(gb + k2 * _LN, _LN)] + bv
                            vs, _ = plsc.sort_key_val(vb, lane,
                                                      descending=False)

                            @pl.when(vs[_LN - 1] > thv[...][0])
                            def _do_merge():
                                fb = (jt * _V + voff + gb + k2 * _LN)
                                merge(vb, fb)
                            return c3
                        lax.fori_loop(0, 8, vec_one, 0)
                    return c2
                lax.fori_loop(0, 8, grp, 0)
            return carry
        lax.fori_loop(0, 8, sub, 0)

    copy(0, buf0, sem0).start()
    copy(1, buf1, sem1).start()

    def pair(i, carry):
        t0 = i * 2
        copy(t0, buf0, sem0).wait()
        scan_chunk(t0, buf0)
        copy(jnp.minimum(t0 + 2, _NCH - 1), buf0, sem0).start()
        copy(t0 + 1, buf1, sem1).wait()
        scan_chunk(t0 + 1, buf1)
        copy(jnp.minimum(t0 + 3, _NCH - 1), buf1, sem1).start()
        return carry

    lax.fori_loop(0, _NCH // 2, pair, 0)
    copy(_NCH - 1, buf0, sem0).wait()
    copy(_NCH - 1, buf1, sem1).wait()

    outv = lax.rev(tvv[...], (0,))
    outi = lax.rev(tiv[...], (0,))
    bix = outi // _V
    sel = outi - bix * _V
    src = bix + w * _NB
    stg_v[...] = outv
    stg_src[...] = src
    stg_sel[...] = sel
    pltpu.sync_copy(stg_v.at[pl.ds(0, 8)], osum_ref.at[pl.ds(w * _NB, 8)])
    pltpu.sync_copy(stg_src.at[pl.ds(0, 8)], osrc_ref.at[pl.ds(w * _NB, 8)])
    pltpu.sync_copy(stg_sel.at[pl.ds(0, 8)], osel_ref.at[pl.ds(w * _NB, 8)])


def _sc_topk(logprobs, bias_flat):
    n = logprobs.shape[0]
    kern = functools.partial(
        pl.kernel,
        mesh=plsc.VectorSubcoreMesh(core_axis_name="c", subcore_axis_name="s"),
        compiler_params=pltpu.CompilerParams(needs_layout_passes=False),
        out_type=[
            jax.ShapeDtypeStruct((n,), jnp.float32),
            jax.ShapeDtypeStruct((n,), jnp.int32),
            jax.ShapeDtypeStruct((n,), jnp.int32),
        ],
        scratch_types=[
            pltpu.VMEM((_CH,), jnp.float32),
            pltpu.VMEM((_CH,), jnp.float32),
            pltpu.VMEM((_NB * _LN,), jnp.float32),
            pltpu.VMEM((_LN,), jnp.float32),
            pltpu.VMEM((_LN,), jnp.int32),
            pltpu.VMEM((_LN,), jnp.float32),
            pltpu.VMEM((_LN,), jnp.float32),
            pltpu.VMEM((_LN,), jnp.int32),
            pltpu.VMEM((_LN,), jnp.int32),
            pltpu.SemaphoreType.DMA,
            pltpu.SemaphoreType.DMA,
        ],
    )(_sc_topk_body)
    return kern(logprobs, bias_flat)


def _gather_body(src_ref, sel_ref, slp_ref, lp_ref, st_ref, seq_ref,
                 oslp_ref, ost_ref, oseq_ref):
    t = slp_ref.shape[1]
    nb = oslp_ref.shape[2]
    i = pl.program_id(0)
    jj = i % nb
    bix = src_ref[i] - (i // nb) * nb
    for tt in range(t):
        oslp_ref[0, tt, pl.ds(jj, 1), :] = slp_ref[0, pl.ds(tt, 1), :]
    oslp_ref[0, t, pl.ds(jj, 1), :] = lp_ref[pl.ds(bix, 1), :]
    ost_ref[:, 0, pl.ds(jj, 1), :] = st_ref[:, 0, pl.ds(bix, 1), :]
    oseq_ref[0, pl.ds(jj, 1), 0:t] = seq_ref[0, pl.ds(bix, 1), :]
    oseq_ref[0, pl.ds(jj, 1), t:t + 1] = jnp.full((1, 1), sel_ref[i],
                                                  oseq_ref.dtype)


def kernel(logprobs, beam_logprobs_sum, beam_seq, beam_seq_logprobs, state):
    B, BD = beam_logprobs_sum.shape
    V = logprobs.shape[-1]
    T = beam_seq.shape[-1]
    L, R, D = state.shape

    bias_flat = jnp.broadcast_to(
        beam_logprobs_sum[:, :, None], (B, BD, _LN)).reshape(-1)
    sums, srcflat, selflat = _sc_topk(logprobs, bias_flat)

    slp4 = beam_seq_logprobs.reshape(B * BD, T, V)
    st8 = state.reshape(L, B, BD, D)

    grid_spec = pltpu.PrefetchScalarGridSpec(
        num_scalar_prefetch=2,
        grid=(B * BD,),
        in_specs=[
            pl.BlockSpec((1, T, V), lambda i, s, e: (s[i], 0, 0)),
            pl.BlockSpec((BD, V), lambda i, s, e: (i // BD, 0)),
            pl.BlockSpec((L, 1, BD, D), lambda i, s, e: (0, i // BD, 0, 0)),
            pl.BlockSpec((1, BD, T), lambda i, s, e: (i // BD, 0, 0)),
        ],
        out_specs=[
            pl.BlockSpec((1, T + 1, BD, V), lambda i, s, e: (i // BD, 0, 0, 0)),
            pl.BlockSpec((L, 1, BD, D), lambda i, s, e: (0, i // BD, 0, 0)),
            pl.BlockSpec((1, BD, T + 1), lambda i, s, e: (i // BD, 0, 0)),
        ],
    )
    oslp, ost, oseq = pl.pallas_call(
        _gather_body,
        grid_spec=grid_spec,
        out_shape=[
            jax.ShapeDtypeStruct((B, T + 1, BD, V), jnp.float32),
            jax.ShapeDtypeStruct((L, B, BD, D), jnp.float32),
            jax.ShapeDtypeStruct((B, BD, T + 1), beam_seq.dtype),
        ],
    )(srcflat, selflat, slp4, logprobs, st8, beam_seq)

    return (oseq,
            oslp.transpose(0, 2, 1, 3),
            sums.reshape(B, BD),
            ost.reshape(L, R, D))


# speedup vs baseline: 2.5696x; 1.0872x over previous
import functools

import jax
import jax.numpy as jnp
from jax import lax
from jax.experimental import pallas as pl
from jax.experimental.pallas import tpu as pltpu
from jax.experimental.pallas import tpu_sc as plsc

_NB = 8
_V = 32768
_CW = 4096
_LN = 16
_NCH = 8

_DNUMS = lax.GatherDimensionNumbers(
    offset_dims=(), collapsed_slice_dims=(0,), start_index_map=(0,))


def _sc_topk_body(lp_ref, bias_ref, osum_ref, osrc_ref, osel_ref,
                  buf0, buf1, biasv, tvv, tiv, thv,
                  stg_v, stg_src, stg_sel, sem0, sem1):
    w = lax.axis_index("s") * 2 + lax.axis_index("c")
    lane = lax.iota(jnp.int32, _LN)
    neg = jnp.float32(-jnp.inf)
    eight = jnp.full((_LN,), 8, jnp.int32)

    def lane_splat(vec, idx):
        return lax.gather(vec, idx[:, None], _DNUMS, (1,),
                          mode=lax.GatherScatterMode.PROMISE_IN_BOUNDS)

    tvv[...] = jnp.full((_LN,), neg, jnp.float32)
    tiv[...] = jnp.full((_LN,), 0, jnp.int32)
    thv[...] = jnp.full((_LN,), neg, jnp.float32)

    pltpu.sync_copy(bias_ref.at[pl.ds(w * (_NB * _LN), _NB * _LN)], biasv)

    def copy(c, buf, sem):
        return pltpu.make_async_copy(
            lp_ref.at[w, :, pl.ds(c * _CW, _CW)], buf, sem)

    def bias_vec(s):
        return biasv[pl.ds(s * _LN, _LN)]

    def merge(vb, fb):
        tv = tvv[...]
        ti = tiv[...]
        idx = jnp.full((_LN,), fb, jnp.int32) + lane
        cav, cai = plsc.sort_key_val(vb, idx, descending=False)
        cdv = lax.rev(cav, (0,))
        cdi = lax.rev(cai, (0,))
        lo = lane < 8
        combv = jnp.where(lo, cdv, tv)
        combi = jnp.where(lo, cdi, ti)
        tv2, ti2 = plsc.sort_key_val(combv, combi, descending=False)
        tvv[...] = tv2
        tiv[...] = ti2
        thv[...] = lane_splat(tv2, eight)

    def scan_chunk(c, buf):
        def row(j, carry_j):
            bv = bias_vec(j)

            def sub(g, carry):
                base = g * 1024
                th0 = thv[...][0]
                rm = jnp.full((_LN,), neg, jnp.float32)
                for k in range(64):
                    v = buf[j, pl.ds(base + k * _LN, _LN)]
                    rm = jnp.maximum(rm, v)
                srt, _ = plsc.sort_key_val(rm + bv, lane, descending=False)

                @pl.when(srt[_LN - 1] > th0)
                def _rescan():
                    def grp(g2, c2):
                        gb = base + g2 * 128
                        gm = jnp.full((_LN,), neg, jnp.float32)
                        for k in range(8):
                            gm = jnp.maximum(gm,
                                             buf[j, pl.ds(gb + k * _LN, _LN)])
                        gs, _ = plsc.sort_key_val(gm + bv, lane,
                                                  descending=False)

                        @pl.when(gs[_LN - 1] > thv[...][0])
                        def _grp_scan():
                            def vec_one(k2, c3):
                                vb = buf[j, pl.ds(gb + k2 * _LN, _LN)] + bv
                                vs, _ = plsc.sort_key_val(vb, lane,
                                                          descending=False)

                                @pl.when(vs[_LN - 1] > thv[...][0])
                                def _do_merge():
                                    fb = (j * _V + c * _CW + gb + k2 * _LN)
                                    merge(vb, fb)
                                return c3
                            lax.fori_loop(0, 8, vec_one, 0)
                        return c2
                    lax.fori_loop(0, 8, grp, 0)
                return carry
            lax.fori_loop(0, _CW // 1024, sub, 0)
            return carry_j
        lax.fori_loop(0, _NB, row, 0)

    copy(0, buf0, sem0).start()
    copy(1, buf1, sem1).start()

    def pair(i, carry):
        t0 = i * 2
        copy(t0, buf0, sem0).wait()
        scan_chunk(t0, buf0)
        copy(jnp.minimum(t0 + 2, _NCH - 1), buf0, sem0).start()
        copy(t0 + 1, buf1, sem1).wait()
        scan_chunk(t0 + 1, buf1)
        copy(jnp.minimum(t0 + 3, _NCH - 1), buf1, sem1).start()
        return carry

    lax.fori_loop(0, _NCH // 2, pair, 0)
    copy(_NCH - 1, buf0, sem0).wait()
    copy(_NCH - 1, buf1, sem1).wait()

    outv = lax.rev(tvv[...], (0,))
    outi = lax.rev(tiv[...], (0,))
    bix = outi // _V
    sel = outi - bix * _V
    src = bix + w * _NB
    stg_v[...] = outv
    stg_src[...] = src
    stg_sel[...] = sel
    pltpu.sync_copy(stg_v.at[pl.ds(0, 8)], osum_ref.at[pl.ds(w * _NB, 8)])
    pltpu.sync_copy(stg_src.at[pl.ds(0, 8)], osrc_ref.at[pl.ds(w * _NB, 8)])
    pltpu.sync_copy(stg_sel.at[pl.ds(0, 8)], osel_ref.at[pl.ds(w * _NB, 8)])


def _sc_topk(lp3, bias_flat):
    n = lp3.shape[0] * lp3.shape[1]
    kern = functools.partial(
        pl.kernel,
        mesh=plsc.VectorSubcoreMesh(core_axis_name="c", subcore_axis_name="s"),
        compiler_params=pltpu.CompilerParams(needs_layout_passes=False),
        out_type=[
            jax.ShapeDtypeStruct((n,), jnp.float32),
            jax.ShapeDtypeStruct((n,), jnp.int32),
            jax.ShapeDtypeStruct((n,), jnp.int32),
        ],
        scratch_types=[
            pltpu.VMEM((_NB, _CW), jnp.float32),
            pltpu.VMEM((_NB, _CW), jnp.float32),
            pltpu.VMEM((_NB * _LN,), jnp.float32),
            pltpu.VMEM((_LN,), jnp.float32),
            pltpu.VMEM((_LN,), jnp.int32),
            pltpu.VMEM((_LN,), jnp.float32),
            pltpu.VMEM((_LN,), jnp.float32),
            pltpu.VMEM((_LN,), jnp.int32),
            pltpu.VMEM((_LN,), jnp.int32),
            pltpu.SemaphoreType.DMA,
            pltpu.SemaphoreType.DMA,
        ],
    )(_sc_topk_body)
    return kern(lp3, bias_flat)


def _gather_body(src_ref, sel_ref, slp_ref, lp_ref, st_ref, seq_ref,
                 oslp_ref, ost_ref, oseq_ref):
    t = slp_ref.shape[1]
    nb = oslp_ref.shape[2]
    i = pl.program_id(0)
    jj = i % nb
    bix = src_ref[i] - (i // nb) * nb
    for tt in range(t):
        oslp_ref[0, tt, pl.ds(jj, 1), :] = slp_ref[0, pl.ds(tt, 1), :]
    oslp_ref[0, t, pl.ds(jj, 1), :] = lp_ref[pl.ds(bix, 1), :]
    ost_ref[:, 0, pl.ds(jj, 1), :] = st_ref[:, 0, pl.ds(bix, 1), :]
    oseq_ref[0, pl.ds(jj, 1), 0:t] = seq_ref[0, pl.ds(bix, 1), :]
    oseq_ref[0, pl.ds(jj, 1), t:t + 1] = jnp.full((1, 1), sel_ref[i],
                                                  oseq_ref.dtype)


def kernel(logprobs, beam_logprobs_sum, beam_seq, beam_seq_logprobs, state):
    B, BD = beam_logprobs_sum.shape
    V = logprobs.shape[-1]
    T = beam_seq.shape[-1]
    L, R, D = state.shape

    bias_flat = jnp.broadcast_to(
        beam_logprobs_sum[:, :, None], (B, BD, _LN)).reshape(-1)
    sums, srcflat, selflat = _sc_topk(logprobs.reshape(B, BD, V), bias_flat)

    slp4 = beam_seq_logprobs.reshape(B * BD, T, V)
    st8 = state.reshape(L, B, BD, D)

    grid_spec = pltpu.PrefetchScalarGridSpec(
        num_scalar_prefetch=2,
        grid=(B * BD,),
        in_specs=[
            pl.BlockSpec((1, T, V), lambda i, s, e: (s[i], 0, 0)),
            pl.BlockSpec((BD, V), lambda i, s, e: (i // BD, 0)),
            pl.BlockSpec((L, 1, BD, D), lambda i, s, e: (0, i // BD, 0, 0)),
            pl.BlockSpec((1, BD, T), lambda i, s, e: (i // BD, 0, 0)),
        ],
        out_specs=[
            pl.BlockSpec((1, T + 1, BD, V), lambda i, s, e: (i // BD, 0, 0, 0)),
            pl.BlockSpec((L, 1, BD, D), lambda i, s, e: (0, i // BD, 0, 0)),
            pl.BlockSpec((1, BD, T + 1), lambda i, s, e: (i // BD, 0, 0)),
        ],
    )
    oslp, ost, oseq = pl.pallas_call(
        _gather_body,
        grid_spec=grid_spec,
        out_shape=[
            jax.ShapeDtypeStruct((B, T + 1, BD, V), jnp.float32),
            jax.ShapeDtypeStruct((L, B, BD, D), jnp.float32),
            jax.ShapeDtypeStruct((B, BD, T + 1), beam_seq.dtype),
        ],
    )(srcflat, selflat, slp4, logprobs, st8, beam_seq)

    return (oseq,
            oslp.transpose(0, 2, 1, 3),
            sums.reshape(B, BD),
            ost.reshape(L, R, D))


# speedup vs baseline: 2.5726x; 1.0012x over previous
import functools

import jax
import jax.numpy as jnp
from jax import lax
from jax.experimental import pallas as pl
from jax.experimental.pallas import tpu as pltpu
from jax.experimental.pallas import tpu_sc as plsc

_NB = 8
_V = 32768
_CW = 4096
_LN = 16
_NCH = 8

_DNUMS = lax.GatherDimensionNumbers(
    offset_dims=(), collapsed_slice_dims=(0,), start_index_map=(0,))


def _sc_topk_body(lp_ref, bias_ref, osum_ref, osrc_ref, osel_ref,
                  buf0, buf1, biasv, tvv, tiv, thv,
                  stg_v, stg_src, stg_sel, sem0, sem1):
    w = lax.axis_index("s") * 2 + lax.axis_index("c")
    lane = lax.iota(jnp.int32, _LN)
    neg = jnp.float32(-jnp.inf)
    eight = jnp.full((_LN,), 8, jnp.int32)

    def lane_splat(vec, idx):
        return lax.gather(vec, idx[:, None], _DNUMS, (1,),
                          mode=lax.GatherScatterMode.PROMISE_IN_BOUNDS)

    tvv[...] = jnp.full((_LN,), neg, jnp.float32)
    tiv[...] = jnp.full((_LN,), 0, jnp.int32)
    thv[...] = jnp.full((_LN,), neg, jnp.float32)

    pltpu.sync_copy(bias_ref.at[pl.ds(w * (_NB * _LN), _NB * _LN)], biasv)

    def copy(c, buf, sem):
        return pltpu.make_async_copy(
            lp_ref.at[w, :, pl.ds(c * _CW, _CW)], buf, sem)

    def bias_vec(s):
        return biasv[pl.ds(s * _LN, _LN)]

    def merge(vb, fb):
        tv = tvv[...]
        ti = tiv[...]
        idx = jnp.full((_LN,), fb, jnp.int32) + lane
        cav, cai = plsc.sort_key_val(vb, idx, descending=False)
        cdv = lax.rev(cav, (0,))
        cdi = lax.rev(cai, (0,))
        lo = lane < 8
        combv = jnp.where(lo, cdv, tv)
        combi = jnp.where(lo, cdi, ti)
        tv2, ti2 = plsc.sort_key_val(combv, combi, descending=False)
        tvv[...] = tv2
        tiv[...] = ti2
        thv[...] = lane_splat(tv2, eight)

    def scan_chunk(c, buf):
        def row(j, carry_j):
            bv = bias_vec(j)

            def sub(g, carry):
                base = g * 1024
                th0 = thv[...][0]
                acc = [jnp.full((_LN,), neg, jnp.float32) for _ in range(8)]
                for k in range(64):
                    v = buf[j, pl.ds(base + k * _LN, _LN)]
                    acc[k % 8] = jnp.maximum(acc[k % 8], v)
                rm = jnp.maximum(
                    jnp.maximum(jnp.maximum(acc[0], acc[1]),
                                jnp.maximum(acc[2], acc[3])),
                    jnp.maximum(jnp.maximum(acc[4], acc[5]),
                                jnp.maximum(acc[6], acc[7])))
                srt, _ = plsc.sort_key_val(rm + bv, lane, descending=False)

                @pl.when(srt[_LN - 1] > th0)
                def _rescan():
                    def grp(g2, c2):
                        gb = base + g2 * 128
                        gm = jnp.full((_LN,), neg, jnp.float32)
                        for k in range(8):
                            gm = jnp.maximum(gm,
                                             buf[j, pl.ds(gb + k * _LN, _LN)])
                        gs, _ = plsc.sort_key_val(gm + bv, lane,
                                                  descending=False)

                        @pl.when(gs[_LN - 1] > thv[...][0])
                        def _grp_scan():
                            def vec_one(k2, c3):
                                vb = buf[j, pl.ds(gb + k2 * _LN, _LN)] + bv
                                vs, _ = plsc.sort_key_val(vb, lane,
                                                          descending=False)

                                @pl.when(vs[_LN - 1] > thv[...][0])
                                def _do_merge():
                                    fb = (j * _V + c * _CW + gb + k2 * _LN)
                                    merge(vb, fb)
                                return c3
                            lax.fori_loop(0, 8, vec_one, 0)
                        return c2
                    lax.fori_loop(0, 8, grp, 0)
                return carry
            lax.fori_loop(0, _CW // 1024, sub, 0)
            return carry_j
        lax.fori_loop(0, _NB, row, 0)

    copy(0, buf0, sem0).start()
    copy(1, buf1, sem1).start()

    def pair(i, carry):
        t0 = i * 2
        copy(t0, buf0, sem0).wait()
        scan_chunk(t0, buf0)
        copy(jnp.minimum(t0 + 2, _NCH - 1), buf0, sem0).start()
        copy(t0 + 1, buf1, sem1).wait()
        scan_chunk(t0 + 1, buf1)
        copy(jnp.minimum(t0 + 3, _NCH - 1), buf1, sem1).start()
        return carry

    lax.fori_loop(0, _NCH // 2, pair, 0)
    copy(_NCH - 1, buf0, sem0).wait()
    copy(_NCH - 1, buf1, sem1).wait()

    outv = lax.rev(tvv[...], (0,))
    outi = lax.rev(tiv[...], (0,))
    bix = outi // _V
    sel = outi - bix * _V
    src = bix + w * _NB
    stg_v[...] = outv
    stg_src[...] = src
    stg_sel[...] = sel
    pltpu.sync_copy(stg_v.at[pl.ds(0, 8)], osum_ref.at[pl.ds(w * _NB, 8)])
    pltpu.sync_copy(stg_src.at[pl.ds(0, 8)], osrc_ref.at[pl.ds(w * _NB, 8)])
    pltpu.sync_copy(stg_sel.at[pl.ds(0, 8)], osel_ref.at[pl.ds(w * _NB, 8)])


def _sc_topk(lp3, bias_flat):
    n = lp3.shape[0] * lp3.shape[1]
    kern = functools.partial(
        pl.kernel,
        mesh=plsc.VectorSubcoreMesh(core_axis_name="c", subcore_axis_name="s"),
        compiler_params=pltpu.CompilerParams(needs_layout_passes=False),
        out_type=[
            jax.ShapeDtypeStruct((n,), jnp.float32),
            jax.ShapeDtypeStruct((n,), jnp.int32),
            jax.ShapeDtypeStruct((n,), jnp.int32),
        ],
        scratch_types=[
            pltpu.VMEM((_NB, _CW), jnp.float32),
            pltpu.VMEM((_NB, _CW), jnp.float32),
            pltpu.VMEM((_NB * _LN,), jnp.float32),
            pltpu.VMEM((_LN,), jnp.float32),
            pltpu.VMEM((_LN,), jnp.int32),
            pltpu.VMEM((_LN,), jnp.float32),
            pltpu.VMEM((_LN,), jnp.float32),
            pltpu.VMEM((_LN,), jnp.int32),
            pltpu.VMEM((_LN,), jnp.int32),
            pltpu.SemaphoreType.DMA,
            pltpu.SemaphoreType.DMA,
        ],
    )(_sc_topk_body)
    return kern(lp3, bias_flat)


def _gather_body(src_ref, sel_ref, slp_ref, lp_ref, st_ref, seq_ref,
                 oslp_ref, ost_ref, oseq_ref):
    t = slp_ref.shape[1]
    nb = oslp_ref.shape[2]
    i = pl.program_id(0)
    jj = i % nb
    bix = src_ref[i] - (i // nb) * nb
    for tt in range(t):
        oslp_ref[0, tt, pl.ds(jj, 1), :] = slp_ref[0, pl.ds(tt, 1), :]
    oslp_ref[0, t, pl.ds(jj, 1), :] = lp_ref[pl.ds(bix, 1), :]
    ost_ref[:, 0, pl.ds(jj, 1), :] = st_ref[:, 0, pl.ds(bix, 1), :]
    oseq_ref[0, pl.ds(jj, 1), 0:t] = seq_ref[0, pl.ds(bix, 1), :]
    oseq_ref[0, pl.ds(jj, 1), t:t + 1] = jnp.full((1, 1), sel_ref[i],
                                                  oseq_ref.dtype)


def kernel(logprobs, beam_logprobs_sum, beam_seq, beam_seq_logprobs, state):
    B, BD = beam_logprobs_sum.shape
    V = logprobs.shape[-1]
    T = beam_seq.shape[-1]
    L, R, D = state.shape

    bias_flat = jnp.broadcast_to(
        beam_logprobs_sum[:, :, None], (B, BD, _LN)).reshape(-1)
    sums, srcflat, selflat = _sc_topk(logprobs.reshape(B, BD, V), bias_flat)

    slp4 = beam_seq_logprobs.reshape(B * BD, T, V)
    st8 = state.reshape(L, B, BD, D)

    grid_spec = pltpu.PrefetchScalarGridSpec(
        num_scalar_prefetch=2,
        grid=(B * BD,),
        in_specs=[
            pl.BlockSpec((1, T, V), lambda i, s, e: (s[i], 0, 0)),
            pl.BlockSpec((BD, V), lambda i, s, e: (i // BD, 0)),
            pl.BlockSpec((L, 1, BD, D), lambda i, s, e: (0, i // BD, 0, 0)),
            pl.BlockSpec((1, BD, T), lambda i, s, e: (i // BD, 0, 0)),
        ],
        out_specs=[
            pl.BlockSpec((1, T + 1, BD, V), lambda i, s, e: (i // BD, 0, 0, 0)),
            pl.BlockSpec((L, 1, BD, D), lambda i, s, e: (0, i // BD, 0, 0)),
            pl.BlockSpec((1, BD, T + 1), lambda i, s, e: (i // BD, 0, 0)),
        ],
    )
    oslp, ost, oseq = pl.pallas_call(
        _gather_body,
        grid_spec=grid_spec,
        out_shape=[
            jax.ShapeDtypeStruct((B, T + 1, BD, V), jnp.float32),
            jax.ShapeDtypeStruct((L, B, BD, D), jnp.float32),
            jax.ShapeDtypeStruct((B, BD, T + 1), beam_seq.dtype),
        ],
    )(srcflat, selflat, slp4, logprobs, st8, beam_seq)

    return (oseq,
            oslp.transpose(0, 2, 1, 3),
            sums.reshape(B, BD),
            ost.reshape(L, R, D))


# speedup vs baseline: 3.0727x; 1.1944x over previous
import functools

import jax
import jax.numpy as jnp
from jax import lax
from jax.experimental import pallas as pl
from jax.experimental.pallas import tpu as pltpu
from jax.experimental.pallas import tpu_sc as plsc

_NB = 8
_V = 32768
_CW = 4096
_LN = 16
_NCH = 8

_DNUMS = lax.GatherDimensionNumbers(
    offset_dims=(), collapsed_slice_dims=(0,), start_index_map=(0,))


def _sc_topk_body(lp_ref, bias_ref, osum_ref, osrc_ref, osel_ref,
                  buf0, buf1, biasv, tvv, tiv, thv,
                  stg_v, stg_src, stg_sel, sem0, sem1):
    w = lax.axis_index("s") * 2 + lax.axis_index("c")
    lane = lax.iota(jnp.int32, _LN)
    neg = jnp.float32(-jnp.inf)
    eight = jnp.full((_LN,), 8, jnp.int32)

    def lane_splat(vec, idx):
        return lax.gather(vec, idx[:, None], _DNUMS, (1,),
                          mode=lax.GatherScatterMode.PROMISE_IN_BOUNDS)

    tvv[...] = jnp.full((_LN,), neg, jnp.float32)
    tiv[...] = jnp.full((_LN,), 0, jnp.int32)
    thv[...] = jnp.full((_LN,), neg, jnp.float32)

    pltpu.sync_copy(bias_ref.at[pl.ds(w * (_NB * _LN), _NB * _LN)], biasv)

    def copy(c, buf, sem):
        return pltpu.make_async_copy(
            lp_ref.at[w, :, pl.ds(c * _CW, _CW)], buf, sem)

    def bias_vec(s):
        return biasv[pl.ds(s * _LN, _LN)]

    def merge(vb, fb):
        tv = tvv[...]
        ti = tiv[...]
        idx = jnp.full((_LN,), fb, jnp.int32) + lane
        cav, cai = plsc.sort_key_val(vb, idx, descending=False)
        cdv = lax.rev(cav, (0,))
        cdi = lax.rev(cai, (0,))
        lo = lane < 8
        combv = jnp.where(lo, cdv, tv)
        combi = jnp.where(lo, cdi, ti)
        tv2, ti2 = plsc.sort_key_val(combv, combi, descending=False)
        tvv[...] = tv2
        tiv[...] = ti2
        thv[...] = lane_splat(tv2, eight)

    def scan_chunk(c, buf):
        def row(j, carry_j):
            bv = bias_vec(j)

            def sub(g, carry):
                base = g * 1024
                th0 = thv[...][0]
                acc = [jnp.full((_LN,), neg, jnp.float32) for _ in range(8)]
                for k in range(64):
                    v = buf[j, pl.ds(base + k * _LN, _LN)]
                    acc[k % 8] = jnp.maximum(acc[k % 8], v)
                rm = jnp.maximum(
                    jnp.maximum(jnp.maximum(acc[0], acc[1]),
                                jnp.maximum(acc[2], acc[3])),
                    jnp.maximum(jnp.maximum(acc[4], acc[5]),
                                jnp.maximum(acc[6], acc[7])))
                srt, _ = plsc.sort_key_val(rm + bv, lane, descending=False)

                @pl.when(srt[_LN - 1] > th0)
                def _rescan():
                    def grp(g2, c2):
                        gb = base + g2 * 128
                        gm = jnp.full((_LN,), neg, jnp.float32)
                        for k in range(8):
                            gm = jnp.maximum(gm,
                                             buf[j, pl.ds(gb + k * _LN, _LN)])
                        gs, _ = plsc.sort_key_val(gm + bv, lane,
                                                  descending=False)

                        @pl.when(gs[_LN - 1] > thv[...][0])
                        def _grp_scan():
                            def vec_one(k2, c3):
                                vb = buf[j, pl.ds(gb + k2 * _LN, _LN)] + bv
                                vs, _ = plsc.sort_key_val(vb, lane,
                                                          descending=False)

                                @pl.when(vs[_LN - 1] > thv[...][0])
                                def _do_merge():
                                    fb = (j * _V + c * _CW + gb + k2 * _LN)
                                    merge(vb, fb)
                                return c3
                            lax.fori_loop(0, 8, vec_one, 0)
                        return c2
                    lax.fori_loop(0, 8, grp, 0)
                return carry
            lax.fori_loop(0, _CW // 1024, sub, 0)
            return carry_j
        lax.fori_loop(0, _NB, row, 0)

    copy(0, buf0, sem0).start()
    copy(1, buf1, sem1).start()

    def pair(i, carry):
        t0 = i * 2
        copy(t0, buf0, sem0).wait()
        scan_chunk(t0, buf0)
        copy(jnp.minimum(t0 + 2, _NCH - 1), buf0, sem0).start()
        copy(t0 + 1, buf1, sem1).wait()
        scan_chunk(t0 + 1, buf1)
        copy(jnp.minimum(t0 + 3, _NCH - 1), buf1, sem1).start()
        return carry

    lax.fori_loop(0, _NCH // 2, pair, 0)
    copy(_NCH - 1, buf0, sem0).wait()
    copy(_NCH - 1, buf1, sem1).wait()

    outv = lax.rev(tvv[...], (0,))
    outi = lax.rev(tiv[...], (0,))
    bix = outi // _V
    sel = outi - bix * _V
    src = bix + w * _NB
    stg_v[...] = outv
    stg_src[...] = src
    stg_sel[...] = sel
    pltpu.sync_copy(stg_v.at[pl.ds(0, 8)], osum_ref.at[pl.ds(w * _NB, 8)])
    pltpu.sync_copy(stg_src.at[pl.ds(0, 8)], osrc_ref.at[pl.ds(w * _NB, 8)])
    pltpu.sync_copy(stg_sel.at[pl.ds(0, 8)], osel_ref.at[pl.ds(w * _NB, 8)])


def _sc_topk(lp3, bias_flat):
    n = lp3.shape[0] * lp3.shape[1]
    kern = functools.partial(
        pl.kernel,
        mesh=plsc.VectorSubcoreMesh(core_axis_name="c", subcore_axis_name="s"),
        compiler_params=pltpu.CompilerParams(needs_layout_passes=False),
        out_type=[
            jax.ShapeDtypeStruct((n,), jnp.float32),
            jax.ShapeDtypeStruct((n,), jnp.int32),
            jax.ShapeDtypeStruct((n,), jnp.int32),
        ],
        scratch_types=[
            pltpu.VMEM((_NB, _CW), jnp.float32),
            pltpu.VMEM((_NB, _CW), jnp.float32),
            pltpu.VMEM((_NB * _LN,), jnp.float32),
            pltpu.VMEM((_LN,), jnp.float32),
            pltpu.VMEM((_LN,), jnp.int32),
            pltpu.VMEM((_LN,), jnp.float32),
            pltpu.VMEM((_LN,), jnp.float32),
            pltpu.VMEM((_LN,), jnp.int32),
            pltpu.VMEM((_LN,), jnp.int32),
            pltpu.SemaphoreType.DMA,
            pltpu.SemaphoreType.DMA,
        ],
    )(_sc_topk_body)
    return kern(lp3, bias_flat)


def _gather_body(src_ref, sel_ref, *refs):
    nb = _NB
    slp_refs = refs[:nb]
    lp_ref, st_ref, seq_ref, oslp_ref, ost_ref, oseq_ref = refs[nb:]
    t = slp_refs[0].shape[1]
    i = pl.program_id(0)
    for jj in range(nb):
        bix = src_ref[i * nb + jj] - i * nb
        for tt in range(t):
            oslp_ref[0, tt, pl.ds(jj, 1), :] = slp_refs[jj][0, pl.ds(tt, 1), :]
        oslp_ref[0, t, pl.ds(jj, 1), :] = lp_ref[pl.ds(bix, 1), :]
        ost_ref[:, 0, pl.ds(jj, 1), :] = st_ref[:, 0, pl.ds(bix, 1), :]
        oseq_ref[0, pl.ds(jj, 1), 0:t] = seq_ref[0, pl.ds(bix, 1), :]
        oseq_ref[0, pl.ds(jj, 1), t:t + 1] = jnp.full(
            (1, 1), sel_ref[i * nb + jj], oseq_ref.dtype)


def kernel(logprobs, beam_logprobs_sum, beam_seq, beam_seq_logprobs, state):
    B, BD = beam_logprobs_sum.shape
    V = logprobs.shape[-1]
    T = beam_seq.shape[-1]
    L, R, D = state.shape

    bias_flat = jnp.broadcast_to(
        beam_logprobs_sum[:, :, None], (B, BD, _LN)).reshape(-1)
    sums, srcflat, selflat = _sc_topk(logprobs.reshape(B, BD, V), bias_flat)

    slp4 = beam_seq_logprobs.reshape(B * BD, T, V)
    st8 = state.reshape(L, B, BD, D)

    def _slp_map(jj):
        return lambda i, s, e: (s[i * BD + jj], 0, 0)

    grid_spec = pltpu.PrefetchScalarGridSpec(
        num_scalar_prefetch=2,
        grid=(B,),
        in_specs=(
            [pl.BlockSpec((1, T, V), _slp_map(jj)) for jj in range(BD)] + [
                pl.BlockSpec((BD, V), lambda i, s, e: (i, 0)),
                pl.BlockSpec((L, 1, BD, D), lambda i, s, e: (0, i, 0, 0)),
                pl.BlockSpec((1, BD, T), lambda i, s, e: (i, 0, 0)),
            ]),
        out_specs=[
            pl.BlockSpec((1, T + 1, BD, V), lambda i, s, e: (i, 0, 0, 0)),
            pl.BlockSpec((L, 1, BD, D), lambda i, s, e: (0, i, 0, 0)),
            pl.BlockSpec((1, BD, T + 1), lambda i, s, e: (i, 0, 0)),
        ],
    )
    oslp, ost, oseq = pl.pallas_call(
        _gather_body,
        grid_spec=grid_spec,
        out_shape=[
            jax.ShapeDtypeStruct((B, T + 1, BD, V), jnp.float32),
            jax.ShapeDtypeStruct((L, B, BD, D), jnp.float32),
            jax.ShapeDtypeStruct((B, BD, T + 1), beam_seq.dtype),
        ],
    )(srcflat, selflat, *([slp4] * BD), logprobs, st8, beam_seq)

    return (oseq,
            oslp.transpose(0, 2, 1, 3),
            sums.reshape(B, BD),
            ost.reshape(L, R, D))
